# Initial kernel scaffold; baseline (speedup 1.0000x reference)
import jax
import jax.numpy as jnp
from jax.experimental import pallas as pl


def kernel(hidden_states, weight, e_score_correction_bias):
    raise NotImplementedError("write your pallas kernel here")


# speedup vs baseline: 1.3853x; 1.3853x over previous
import jax
import jax.numpy as jnp
from jax.experimental import pallas as pl

N_EXPERTS = 64
K_TOP = 8
HIDDEN_DIM = 4096
N_GROUPS = 8
GSIZE = 8
TOPK_GROUPS = 4
ROUTE_SCALE = 2.5
TB = 256

_NEG = -1e30


def _router_block(hs_ref, wt_ref, bias_ref, idx_ref, w_ref):
    logits = jnp.dot(
        hs_ref[...].astype(jnp.bfloat16), wt_ref[...].astype(jnp.bfloat16),
        preferred_element_type=jnp.float32,
    )
    scores = jax.nn.sigmoid(logits)
    sfc = scores + bias_ref[...]
    lane = jax.lax.broadcasted_iota(jnp.int32, (TB, N_EXPERTS), 1)

    gparts = []
    for g in range(N_GROUPS):
        ing = (lane // GSIZE) == g
        sg = jnp.where(ing, sfc, _NEG)
        m1 = jnp.max(sg, axis=-1, keepdims=True)
        l1 = jnp.min(jnp.where(sg == m1, lane, N_EXPERTS), axis=-1, keepdims=True)
        m2 = jnp.max(jnp.where(lane == l1, _NEG, sg), axis=-1, keepdims=True)
        gparts.append(m1 + m2)
    gs = jnp.concatenate(gparts, axis=-1)

    glane = jax.lax.broadcasted_iota(jnp.int32, (TB, N_GROUPS), 1)
    gmask = jnp.zeros((TB, N_GROUPS), jnp.float32)
    gwork = gs
    for _ in range(TOPK_GROUPS):
        gm = jnp.max(gwork, axis=-1, keepdims=True)
        gl = jnp.min(jnp.where(gwork == gm, glane, N_GROUPS), axis=-1, keepdims=True)
        sel = glane == gl
        gmask = jnp.where(sel, 1.0, gmask)
        gwork = jnp.where(sel, _NEG, gwork)

    smask = jnp.concatenate(
        [jnp.broadcast_to(gmask[:, g:g + 1], (TB, GSIZE)) for g in range(N_GROUPS)],
        axis=-1,
    )
    ms = jnp.where(smask > 0.0, sfc, 0.0)

    idxs, ws = [], []
    for _ in range(K_TOP):
        m = jnp.max(ms, axis=-1, keepdims=True)
        l = jnp.min(jnp.where(ms == m, lane, N_EXPERTS), axis=-1, keepdims=True)
        sel = lane == l
        w = jnp.sum(jnp.where(sel, scores, 0.0), axis=-1, keepdims=True)
        idxs.append(l)
        ws.append(w)
        ms = jnp.where(sel, _NEG, ms)
    tidx = jnp.concatenate(idxs, axis=-1)
    tw = jnp.concatenate(ws, axis=-1)
    tw = tw / (jnp.sum(tw, axis=-1, keepdims=True) + 1e-20) * ROUTE_SCALE

    idx_ref[...] = tidx
    w_ref[...] = tw


def kernel(hidden_states, weight, e_score_correction_bias):
    hs = hidden_states.reshape(-1, hidden_states.shape[-1]).astype(jnp.float32)
    tokens = hs.shape[0]
    wt = weight.astype(jnp.float32).T
    bias = e_score_correction_bias.astype(jnp.float32).reshape(1, N_EXPERTS)
    grid = (tokens // TB,)
    tidx, tw = pl.pallas_call(
        _router_block,
        grid=grid,
        in_specs=[
            pl.BlockSpec((TB, HIDDEN_DIM), lambda i: (i, 0)),
            pl.BlockSpec((HIDDEN_DIM, N_EXPERTS), lambda i: (0, 0)),
            pl.BlockSpec((1, N_EXPERTS), lambda i: (0, 0)),
        ],
        out_specs=[
            pl.BlockSpec((TB, K_TOP), lambda i: (i, 0)),
            pl.BlockSpec((TB, K_TOP), lambda i: (i, 0)),
        ],
        out_shape=[
            jax.ShapeDtypeStruct((tokens, K_TOP), jnp.int32),
            jax.ShapeDtypeStruct((tokens, K_TOP), jnp.float32),
        ],
    )(hs, wt, bias)
    return tidx, tw


# speedup vs baseline: 3.5947x; 2.5948x over previous
import jax
import jax.numpy as jnp
from jax.experimental import pallas as pl

N_EXPERTS = 64
K_TOP = 8
HIDDEN_DIM = 4096
N_GROUPS = 8
GSIZE = 8
TOPK_GROUPS = 4
ROUTE_SCALE = 2.5
TB = 256

_NEG = -1e30


def _router_block(hs_ref, wt_ref, bias_ref, idx_ref, w_ref):
    logits = jnp.dot(
        hs_ref[...].astype(jnp.bfloat16), wt_ref[...].astype(jnp.bfloat16),
        preferred_element_type=jnp.float32,
    )
    lt = logits.T
    scores = jax.nn.sigmoid(lt)
    sfc = scores + bias_ref[...]
    eidx = jax.lax.broadcasted_iota(
        jnp.int32, (N_EXPERTS, TB), 0).astype(jnp.float32)

    gparts = []
    for g in range(N_GROUPS):
        sg = sfc[g * GSIZE:(g + 1) * GSIZE, :]
        m1 = jnp.max(sg, axis=0, keepdims=True)
        m2 = jnp.max(jnp.where(sg == m1, _NEG, sg), axis=0, keepdims=True)
        gparts.append(m1 + m2)
    gs = jnp.concatenate(gparts, axis=0)

    gidx = jax.lax.broadcasted_iota(
        jnp.int32, (N_GROUPS, TB), 0).astype(jnp.float32)
    gmask = jnp.zeros((N_GROUPS, TB), jnp.float32)
    gwork = gs
    for _ in range(TOPK_GROUPS):
        gm = jnp.max(gwork, axis=0, keepdims=True)
        gl = jnp.min(jnp.where(gwork == gm, gidx, float(N_GROUPS)),
                     axis=0, keepdims=True)
        sel = gidx == gl
        gmask = jnp.where(sel, 1.0, gmask)
        gwork = jnp.where(sel, _NEG, gwork)

    smask = jnp.concatenate(
        [jnp.broadcast_to(gmask[g:g + 1, :], (GSIZE, TB))
         for g in range(N_GROUPS)], axis=0)
    ms = jnp.where(smask > 0.0, sfc, 0.0)

    idxs, ws = [], []
    for _ in range(K_TOP):
        m = jnp.max(ms, axis=0, keepdims=True)
        l = jnp.min(jnp.where(ms == m, eidx, float(N_EXPERTS)),
                    axis=0, keepdims=True)
        sel = eidx == l
        w = jnp.sum(jnp.where(sel, scores, 0.0), axis=0, keepdims=True)
        idxs.append(l)
        ws.append(w)
        ms = jnp.where(sel, _NEG, ms)
    tidx_t = jnp.concatenate(idxs, axis=0)
    tw_t = jnp.concatenate(ws, axis=0)
    tw_t = tw_t / (jnp.sum(tw_t, axis=0, keepdims=True) + 1e-20) * ROUTE_SCALE

    idx_ref[...] = tidx_t.T.astype(jnp.int32)
    w_ref[...] = tw_t.T


def kernel(hidden_states, weight, e_score_correction_bias):
    hs = hidden_states.reshape(-1, hidden_states.shape[-1]).astype(jnp.float32)
    tokens = hs.shape[0]
    wt = weight.astype(jnp.float32).T
    bias = e_score_correction_bias.astype(jnp.float32).reshape(N_EXPERTS, 1)
    grid = (tokens // TB,)
    tidx, tw = pl.pallas_call(
        _router_block,
        grid=grid,
        in_specs=[
            pl.BlockSpec((TB, HIDDEN_DIM), lambda i: (i, 0)),
            pl.BlockSpec((HIDDEN_DIM, N_EXPERTS), lambda i: (0, 0)),
            pl.BlockSpec((N_EXPERTS, 1), lambda i: (0, 0)),
        ],
        out_specs=[
            pl.BlockSpec((TB, K_TOP), lambda i: (i, 0)),
            pl.BlockSpec((TB, K_TOP), lambda i: (i, 0)),
        ],
        out_shape=[
            jax.ShapeDtypeStruct((tokens, K_TOP), jnp.int32),
            jax.ShapeDtypeStruct((tokens, K_TOP), jnp.float32),
        ],
    )(hs, wt, bias)
    return tidx, tw


# speedup vs baseline: 4.3544x; 1.2113x over previous
import jax
import jax.numpy as jnp
from jax.experimental import pallas as pl

N_EXPERTS = 64
K_TOP = 8
HIDDEN_DIM = 4096
N_GROUPS = 8
GSIZE = 8
TOPK_GROUPS = 4
ROUTE_SCALE = 2.5
TB = 512

_NEG = -1e30


def _router_block(hs_ref, wt_ref, bias_ref, idx_ref, w_ref):
    logits = jnp.dot(
        hs_ref[...].astype(jnp.bfloat16), wt_ref[...].astype(jnp.bfloat16),
        preferred_element_type=jnp.float32,
    )
    lt = logits.T
    scores = jax.nn.sigmoid(lt)
    sfc = scores + bias_ref[...]
    eidx = jax.lax.broadcasted_iota(
        jnp.int32, (N_EXPERTS, TB), 0).astype(jnp.float32)

    gparts = []
    for g in range(N_GROUPS):
        sg = sfc[g * GSIZE:(g + 1) * GSIZE, :]
        m1 = jnp.max(sg, axis=0, keepdims=True)
        m2 = jnp.max(jnp.where(sg == m1, _NEG, sg), axis=0, keepdims=True)
        gparts.append(m1 + m2)
    gs = jnp.concatenate(gparts, axis=0)

    gidx = jax.lax.broadcasted_iota(
        jnp.int32, (N_GROUPS, TB), 0).astype(jnp.float32)
    gmask = jnp.zeros((N_GROUPS, TB), jnp.float32)
    gwork = gs
    for _ in range(TOPK_GROUPS):
        gm = jnp.max(gwork, axis=0, keepdims=True)
        gl = jnp.min(jnp.where(gwork == gm, gidx, float(N_GROUPS)),
                     axis=0, keepdims=True)
        sel = gidx == gl
        gmask = jnp.where(sel, 1.0, gmask)
        gwork = jnp.where(sel, _NEG, gwork)

    smask = jnp.concatenate(
        [jnp.broadcast_to(gmask[g:g + 1, :], (GSIZE, TB))
         for g in range(N_GROUPS)], axis=0)
    ms = jnp.where(smask > 0.0, sfc, 0.0)

    idxs, ws = [], []
    for _ in range(K_TOP):
        m = jnp.max(ms, axis=0, keepdims=True)
        l = jnp.min(jnp.where(ms == m, eidx, float(N_EXPERTS)),
                    axis=0, keepdims=True)
        sel = eidx == l
        w = jnp.sum(jnp.where(sel, scores, 0.0), axis=0, keepdims=True)
        idxs.append(l)
        ws.append(w)
        ms = jnp.where(sel, _NEG, ms)
    tidx_t = jnp.concatenate(idxs, axis=0)
    tw_t = jnp.concatenate(ws, axis=0)
    tw_t = tw_t / (jnp.sum(tw_t, axis=0, keepdims=True) + 1e-20) * ROUTE_SCALE

    idx_ref[...] = tidx_t.T.astype(jnp.int32)
    w_ref[...] = tw_t.T


def kernel(hidden_states, weight, e_score_correction_bias):
    hs = hidden_states.reshape(-1, hidden_states.shape[-1]).astype(jnp.float32)
    tokens = hs.shape[0]
    wt = weight.astype(jnp.float32).T
    bias = e_score_correction_bias.astype(jnp.float32).reshape(N_EXPERTS, 1)
    grid = (tokens // TB,)
    tidx, tw = pl.pallas_call(
        _router_block,
        grid=grid,
        in_specs=[
            pl.BlockSpec((TB, HIDDEN_DIM), lambda i: (i, 0)),
            pl.BlockSpec((HIDDEN_DIM, N_EXPERTS), lambda i: (0, 0)),
            pl.BlockSpec((N_EXPERTS, 1), lambda i: (0, 0)),
        ],
        out_specs=[
            pl.BlockSpec((TB, K_TOP), lambda i: (i, 0)),
            pl.BlockSpec((TB, K_TOP), lambda i: (i, 0)),
        ],
        out_shape=[
            jax.ShapeDtypeStruct((tokens, K_TOP), jnp.int32),
            jax.ShapeDtypeStruct((tokens, K_TOP), jnp.float32),
        ],
    )(hs, wt, bias)
    return tidx, tw


# speedup vs baseline: 4.4882x; 1.0307x over previous
import jax
import jax.numpy as jnp
from jax.experimental import pallas as pl

N_EXPERTS = 64
K_TOP = 8
HIDDEN_DIM = 4096
N_GROUPS = 8
GSIZE = 8
TOPK_GROUPS = 4
ROUTE_SCALE = 2.5
TB = 512

_NEG = -1e30


def _router_block(hs_ref, wt_ref, bias_ref, idx_ref, w_ref):
    logits = jax.lax.dot_general(
        hs_ref[...], wt_ref[...],
        dimension_numbers=(((1,), (1,)), ((), ())),
        preferred_element_type=jnp.float32,
    )
    lt = logits.T
    scores = jax.nn.sigmoid(lt)
    sfc = scores + bias_ref[...]
    eidx = jax.lax.broadcasted_iota(
        jnp.int32, (N_EXPERTS, TB), 0).astype(jnp.float32)

    gparts = []
    for g in range(N_GROUPS):
        sg = sfc[g * GSIZE:(g + 1) * GSIZE, :]
        m1 = jnp.max(sg, axis=0, keepdims=True)
        m2 = jnp.max(jnp.where(sg == m1, _NEG, sg), axis=0, keepdims=True)
        gparts.append(m1 + m2)
    gs = jnp.concatenate(gparts, axis=0)

    gidx = jax.lax.broadcasted_iota(
        jnp.int32, (N_GROUPS, TB), 0).astype(jnp.float32)
    gmask = jnp.zeros((N_GROUPS, TB), jnp.float32)
    gwork = gs
    for _ in range(TOPK_GROUPS):
        gm = jnp.max(gwork, axis=0, keepdims=True)
        gl = jnp.min(jnp.where(gwork == gm, gidx, float(N_GROUPS)),
                     axis=0, keepdims=True)
        sel = gidx == gl
        gmask = jnp.where(sel, 1.0, gmask)
        gwork = jnp.where(sel, _NEG, gwork)

    smask = jnp.concatenate(
        [jnp.broadcast_to(gmask[g:g + 1, :], (GSIZE, TB))
         for g in range(N_GROUPS)], axis=0)
    ms = jnp.where(smask > 0.0, sfc, 0.0)

    idxs, ws = [], []
    for _ in range(K_TOP):
        m = jnp.max(ms, axis=0, keepdims=True)
        l = jnp.min(jnp.where(ms == m, eidx, float(N_EXPERTS)),
                    axis=0, keepdims=True)
        sel = eidx == l
        w = jnp.sum(jnp.where(sel, scores, 0.0), axis=0, keepdims=True)
        idxs.append(l)
        ws.append(w)
        ms = jnp.where(sel, _NEG, ms)
    tidx_t = jnp.concatenate(idxs, axis=0)
    tw_t = jnp.concatenate(ws, axis=0)
    tw_t = tw_t / (jnp.sum(tw_t, axis=0, keepdims=True) + 1e-20) * ROUTE_SCALE

    idx_ref[...] = tidx_t.T.astype(jnp.int32)
    w_ref[...] = tw_t.T


def kernel(hidden_states, weight, e_score_correction_bias):
    hs = hidden_states.reshape(-1, hidden_states.shape[-1]).astype(jnp.float32)
    tokens = hs.shape[0]
    wt = weight.astype(jnp.float32)
    bias = e_score_correction_bias.astype(jnp.float32).reshape(N_EXPERTS, 1)
    grid = (tokens // TB,)
    tidx, tw = pl.pallas_call(
        _router_block,
        grid=grid,
        in_specs=[
            pl.BlockSpec((TB, HIDDEN_DIM), lambda i: (i, 0)),
            pl.BlockSpec((N_EXPERTS, HIDDEN_DIM), lambda i: (0, 0)),
            pl.BlockSpec((N_EXPERTS, 1), lambda i: (0, 0)),
        ],
        out_specs=[
            pl.BlockSpec((TB, K_TOP), lambda i: (i, 0)),
            pl.BlockSpec((TB, K_TOP), lambda i: (i, 0)),
        ],
        out_shape=[
            jax.ShapeDtypeStruct((tokens, K_TOP), jnp.int32),
            jax.ShapeDtypeStruct((tokens, K_TOP), jnp.float32),
        ],
    )(hs, wt, bias)
    return tidx, tw
